# baseline (speedup 1.0000x reference)
import functools

import jax
import jax.numpy as jnp
from jax import lax
from jax.experimental import pallas as pl
from jax.experimental.pallas import tpu as pltpu
from jax.experimental.pallas import tpu_sc as plsc

N_CONFIGS = 26
MAX_CLUSTERS = 100000
EMBED_DIM = 32
BATCH = 16384

NC, NS = 2, 16
NW = NC * NS
N_ROWS = N_CONFIGS * EMBED_DIM
ROWS_PER_W = N_ROWS // NW
N_SUPER = ROWS_PER_W // 2
LANES = 16

S0 = 33408
S1 = 33408
S2_MAIN = 33152
TAIL = MAX_CLUSTERS - S0 - S1 - S2_MAIN
S2 = S2_MAIN + TAIL
REGIONS = ((0, S0), (S0, S1), (S0 + S1, S2))
BUF = max(S0, S1, S2)


def _make_kernel():
    mesh = plsc.VectorSubcoreMesh(core_axis_name="c", subcore_axis_name="s")

    @functools.partial(
        pl.kernel,
        out_type=jax.ShapeDtypeStruct((N_ROWS, BATCH), jnp.float32),
        mesh=mesh,
        compiler_params=pltpu.CompilerParams(needs_layout_passes=False),
        scratch_types=[
            pltpu.VMEM((BUF,), jnp.float32),
            pltpu.VMEM((BUF,), jnp.float32),
            pltpu.VMEM((BATCH,), jnp.int32),
            pltpu.VMEM((BATCH,), jnp.float32),
            pltpu.VMEM((BATCH,), jnp.float32),
            pltpu.SemaphoreType.DMA,
            pltpu.SemaphoreType.DMA,
            pltpu.SemaphoreType.DMA,
        ],
    )
    def gather_kernel(t2_hbm, tails_hbm, idx_hbm, out_hbm,
                      reg0_v, reg1_v, idx_v, out0_v, out1_v,
                      rsem0, rsem1, wsem):
        regs = (reg0_v, reg1_v)
        outs = (out0_v, out1_v)
        wid = lax.axis_index("s") * NC + lax.axis_index("c")
        base = wid * ROWS_PER_W
        rsems = (rsem0, rsem1)
        iota = lax.iota(jnp.int32, LANES)

        def issue_region(j, r, p):
            off, _ = REGIONS[j]
            if j < 2:
                pltpu.async_copy(t2_hbm.at[r, pl.ds(off, REGIONS[j][1])],
                                 regs[p].at[pl.ds(0, REGIONS[j][1])],
                                 rsems[p])
            else:
                pltpu.async_copy(t2_hbm.at[r, pl.ds(off, S2_MAIN)],
                                 regs[p].at[pl.ds(0, S2_MAIN)], rsems[p])
                pltpu.async_copy(tails_hbm.at[r],
                                 regs[p].at[pl.ds(S2_MAIN, 128)], rsems[p])

        def wait_region(j, r, p):
            if j < 2:
                pltpu.make_async_copy(t2_hbm.at[r, pl.ds(0, REGIONS[j][1])],
                                      regs[p].at[pl.ds(0, REGIONS[j][1])],
                                      rsems[p]).wait()
            else:
                pltpu.make_async_copy(t2_hbm.at[r, pl.ds(0, S2_MAIN)],
                                      regs[p].at[pl.ds(0, S2_MAIN)],
                                      rsems[p]).wait()
                pltpu.make_async_copy(tails_hbm.at[r],
                                      regs[p].at[pl.ds(S2_MAIN, 128)],
                                      rsems[p]).wait()

        def gather_pass(j, p, ob):
            off, size = REGIONS[j]

            @plsc.parallel_loop(0, BATCH, step=LANES, unroll=16)
            def _(o):
                iv = idx_v[pl.ds(o, LANES)]
                rel = plsc.bitcast(iv - off, jnp.uint32)
                inb = rel < jnp.uint32(size)
                gidx = plsc.bitcast(
                    jnp.minimum(rel, jnp.uint32(size - 1)), jnp.int32)
                val = plsc.load_gather(regs[p], [gidx])
                plsc.store_scatter(outs[ob], [o + iota], val, mask=inb)

        def superstep(s, prev_cfg):
            ra = base + 2 * s
            rb = ra + 1
            rn = jnp.minimum(ra + 2, N_ROWS - 1)
            cfg_a = lax.shift_right_logical(ra, 5)
            cfg_b = lax.shift_right_logical(rb, 5)

            @pl.when(jnp.logical_or(s == 0, cfg_a != prev_cfg))
            def _():
                pltpu.sync_copy(idx_hbm.at[cfg_a], idx_v)

            @pl.when(s > 0)
            def _():
                pltpu.make_async_copy(out0_v, out_hbm.at[ra], wsem).wait()

            wait_region(0, ra, 0)
            gather_pass(0, 0, 0)
            issue_region(2, ra, 0)
            wait_region(1, ra, 1)
            gather_pass(1, 1, 0)
            issue_region(0, rb, 1)
            wait_region(2, ra, 0)
            gather_pass(2, 0, 0)
            issue_region(1, rb, 0)
            pltpu.async_copy(out0_v, out_hbm.at[ra], wsem)

            @pl.when(cfg_b != cfg_a)
            def _():
                pltpu.sync_copy(idx_hbm.at[cfg_b], idx_v)

            @pl.when(s > 0)
            def _():
                pltpu.make_async_copy(out1_v, out_hbm.at[rb], wsem).wait()

            wait_region(0, rb, 1)
            gather_pass(0, 1, 1)
            issue_region(2, rb, 1)
            wait_region(1, rb, 0)
            gather_pass(1, 0, 1)
            issue_region(0, rn, 0)
            wait_region(2, rb, 1)
            gather_pass(2, 1, 1)
            issue_region(1, rn, 1)
            pltpu.async_copy(out1_v, out_hbm.at[rb], wsem)

            return cfg_b

        issue_region(0, base, 0)
        issue_region(1, base, 1)
        lax.fori_loop(0, N_SUPER, superstep, jnp.int32(-1))

        last = jnp.minimum(base + ROWS_PER_W, N_ROWS - 1)
        wait_region(0, last, 0)
        wait_region(1, last, 1)
        for ob_v in (out0_v, out1_v):
            pltpu.make_async_copy(ob_v, out_hbm.at[base], wsem).wait()

    return gather_kernel


_GATHER = _make_kernel()


def kernel(cluster_assignments, tables):
    t2 = jnp.transpose(tables, (0, 2, 1)).reshape(N_ROWS, MAX_CLUSTERS)
    tails = jnp.pad(
        lax.slice(t2, (0, S0 + S1 + S2_MAIN), (N_ROWS, MAX_CLUSTERS)),
        ((0, 0), (0, 128 - TAIL)))
    idx_t = jnp.transpose(cluster_assignments)
    out_t = _GATHER(t2, tails, idx_t)
    return jnp.transpose(out_t.reshape(N_CONFIGS, EMBED_DIM, BATCH),
                         (2, 0, 1))


# speedup vs baseline: 1.0555x; 1.0555x over previous
import functools

import jax
import jax.numpy as jnp
from jax import lax
from jax.experimental import pallas as pl
from jax.experimental.pallas import tpu as pltpu
from jax.experimental.pallas import tpu_sc as plsc

N_CONFIGS = 26
MAX_CLUSTERS = 100000
EMBED_DIM = 32
BATCH = 16384

NC, NS = 2, 16
NW = NC * NS
N_ROWS = N_CONFIGS * EMBED_DIM
ROWS_PER_W = N_ROWS // NW
LANES = 16
OUT_CHUNK = 4096
N_OUT_CHUNK = BATCH // OUT_CHUNK


def _make_kernel():
    mesh = plsc.VectorSubcoreMesh(core_axis_name="c", subcore_axis_name="s")

    @functools.partial(
        pl.kernel,
        out_type=jax.ShapeDtypeStruct((N_ROWS, BATCH), jnp.float32),
        mesh=mesh,
        compiler_params=pltpu.CompilerParams(needs_layout_passes=False),
        scratch_types=[
            pltpu.VMEM((MAX_CLUSTERS,), jnp.float32),
            pltpu.VMEM((BATCH,), jnp.int32),
            pltpu.VMEM((2, OUT_CHUNK), jnp.float32),
            pltpu.SemaphoreType.DMA,
            pltpu.SemaphoreType.DMA,
        ],
    )
    def gather_kernel(t2_hbm, idx_hbm, out_hbm, row_v, idx_v, out_v,
                      wsem0, wsem1):
        wid = lax.axis_index("s") * NC + lax.axis_index("c")
        base = wid * ROWS_PER_W
        wsems = (wsem0, wsem1)

        def rowstep(k, prev_cfg):
            r = base + k
            cfg = lax.shift_right_logical(r, 5)

            @pl.when(jnp.logical_or(k == 0, cfg != prev_cfg))
            def _():
                pltpu.sync_copy(idx_hbm.at[cfg], idx_v)

            pltpu.sync_copy(t2_hbm.at[r], row_v)

            for c in range(N_OUT_CHUNK):
                b = c % 2
                drain = pltpu.make_async_copy(
                    out_v.at[b],
                    out_hbm.at[r, pl.ds(c * OUT_CHUNK, OUT_CHUNK)],
                    wsems[b])
                if c < 2:
                    @pl.when(k > 0)
                    def _():
                        drain.wait()
                else:
                    drain.wait()

                @plsc.parallel_loop(0, OUT_CHUNK, step=LANES, unroll=8)
                def g(o):
                    out_v[b, pl.ds(o, LANES)] = plsc.load_gather(
                        row_v, [idx_v[pl.ds(c * OUT_CHUNK + o, LANES)]])

                pltpu.async_copy(
                    out_v.at[b],
                    out_hbm.at[r, pl.ds(c * OUT_CHUNK, OUT_CHUNK)],
                    wsems[b])
            return cfg

        lax.fori_loop(0, ROWS_PER_W, rowstep, jnp.int32(-1))

        for b in range(2):
            pltpu.make_async_copy(
                out_v.at[b], out_hbm.at[base, pl.ds(0, OUT_CHUNK)],
                wsems[b]).wait()

    return gather_kernel


_GATHER = _make_kernel()


def kernel(cluster_assignments, tables):
    t2 = jnp.transpose(tables, (0, 2, 1)).reshape(N_ROWS, MAX_CLUSTERS)
    idx_t = jnp.transpose(cluster_assignments)
    out_t = _GATHER(t2, idx_t)
    return jnp.transpose(out_t.reshape(N_CONFIGS, EMBED_DIM, BATCH),
                         (2, 0, 1))


# speedup vs baseline: 1.1151x; 1.0564x over previous
import functools

import jax
import jax.numpy as jnp
from jax import lax
from jax.experimental import pallas as pl
from jax.experimental.pallas import tpu as pltpu
from jax.experimental.pallas import tpu_sc as plsc

N_CONFIGS = 26
MAX_CLUSTERS = 100000
EMBED_DIM = 32
BATCH = 16384

NC, NS = 2, 16
NW = NC * NS
N_ROWS = N_CONFIGS * EMBED_DIM
ROWS_PER_W = N_ROWS // NW
LANES = 16
OUT_CHUNK = 2048
N_OUT_CHUNK = BATCH // OUT_CHUNK


def _make_kernel():
    mesh = plsc.VectorSubcoreMesh(core_axis_name="c", subcore_axis_name="s")

    @functools.partial(
        pl.kernel,
        out_type=jax.ShapeDtypeStruct((N_ROWS, BATCH), jnp.float32),
        mesh=mesh,
        compiler_params=pltpu.CompilerParams(needs_layout_passes=False),
        scratch_types=[
            pltpu.VMEM((MAX_CLUSTERS,), jnp.float32),
            pltpu.VMEM((BATCH,), jnp.int32),
            pltpu.VMEM((2, OUT_CHUNK), jnp.float32),
            pltpu.SemaphoreType.DMA,
            pltpu.SemaphoreType.DMA,
        ],
    )
    def gather_kernel(t2_hbm, idx_hbm, out_hbm, row_v, idx_v, out_v,
                      wsem0, wsem1):
        wid = lax.axis_index("s") * NC + lax.axis_index("c")
        base = wid * ROWS_PER_W
        wsems = (wsem0, wsem1)

        def rowstep(k, prev_cfg):
            r = base + k
            cfg = lax.shift_right_logical(r, 5)

            @pl.when(jnp.logical_or(k == 0, cfg != prev_cfg))
            def _():
                pltpu.sync_copy(idx_hbm.at[cfg], idx_v)

            pltpu.sync_copy(t2_hbm.at[r], row_v)

            for c in range(N_OUT_CHUNK):
                b = c % 2
                drain = pltpu.make_async_copy(
                    out_v.at[b],
                    out_hbm.at[r, pl.ds(c * OUT_CHUNK, OUT_CHUNK)],
                    wsems[b])
                if c < 2:
                    @pl.when(k > 0)
                    def _():
                        drain.wait()
                else:
                    drain.wait()

                @plsc.parallel_loop(0, OUT_CHUNK, step=LANES, unroll=8)
                def g(o):
                    out_v[b, pl.ds(o, LANES)] = plsc.load_gather(
                        row_v, [idx_v[pl.ds(c * OUT_CHUNK + o, LANES)]])

                pltpu.async_copy(
                    out_v.at[b],
                    out_hbm.at[r, pl.ds(c * OUT_CHUNK, OUT_CHUNK)],
                    wsems[b])
            return cfg

        lax.fori_loop(0, ROWS_PER_W, rowstep, jnp.int32(-1))

        for b in range(2):
            pltpu.make_async_copy(
                out_v.at[b], out_hbm.at[base, pl.ds(0, OUT_CHUNK)],
                wsems[b]).wait()

    return gather_kernel


_GATHER = _make_kernel()


def kernel(cluster_assignments, tables):
    t2 = jnp.transpose(tables, (0, 2, 1)).reshape(N_ROWS, MAX_CLUSTERS)
    idx_t = jnp.transpose(cluster_assignments)
    out_t = _GATHER(t2, idx_t)
    return jnp.transpose(out_t.reshape(N_CONFIGS, EMBED_DIM, BATCH),
                         (2, 0, 1))
